# baseline (speedup 1.0000x reference)
import functools

import jax
import jax.numpy as jnp
from jax import lax
from jax.experimental import pallas as pl
from jax.experimental.pallas import tpu as pltpu
from jax.experimental.pallas import tpu_sc as plsc

_NC = 2
_NS = 16
_NW = _NC * _NS
_L = 16

_CH = 8


def _sc_mix(n_rows, h):
    rpw = n_rows // _NW
    nch = rpw // _CH
    mesh = plsc.VectorSubcoreMesh(core_axis_name="c", subcore_axis_name="s")

    @functools.partial(
        pl.kernel,
        mesh=mesh,
        out_type=jax.ShapeDtypeStruct((n_rows, h), jnp.float32),
        scratch_types=[
            pltpu.VMEM((2, _CH, h), jnp.float32),
            pltpu.VMEM((2, _CH, h), jnp.float32),
            pltpu.VMEM((2, _CH, h), jnp.float32),
            pltpu.VMEM((rpw * 2 * _L,), jnp.float32),
            pltpu.SemaphoreType.DMA,
            pltpu.SemaphoreType.DMA,
            pltpu.SemaphoreType.DMA,
            pltpu.SemaphoreType.DMA,
        ],
    )
    def mix(e_hbm, w_hbm, out_hbm, e0_v, e1_v, o_v, w_v,
            sin0, sin1, sout0, sout1):
        wid = lax.axis_index("s") * _NC + lax.axis_index("c")
        base = wid * rpw
        sins = (sin0, sin1)
        souts = (sout0, sout1)
        pltpu.sync_copy(w_hbm.at[pl.ds(base * 2 * _L, rpw * 2 * _L)], w_v)

        def start_in(c, b):
            row0 = base + c * _CH
            pltpu.async_copy(e_hbm.at[0, pl.ds(row0, _CH)], e0_v.at[b], sins[b])
            pltpu.async_copy(e_hbm.at[1, pl.ds(row0, _CH)], e1_v.at[b], sins[b])

        def wait_in(c, b):
            row0 = base + c * _CH
            pltpu.make_async_copy(e_hbm.at[0, pl.ds(row0, _CH)], e0_v.at[b], sins[b]).wait()
            pltpu.make_async_copy(e_hbm.at[1, pl.ds(row0, _CH)], e1_v.at[b], sins[b]).wait()

        def compute(c, b):
            def row_body(r, _):
                woff = (c * _CH + r) * 2 * _L
                w0 = w_v[pl.ds(woff, _L)]
                w1 = w_v[pl.ds(woff + _L, _L)]

                def vec_body(j, _):
                    sl = pl.ds(j * _L, _L)
                    o_v[b, r, sl] = e0_v[b, r, sl] * w0 + e1_v[b, r, sl] * w1
                    return 0

                lax.fori_loop(0, h // _L, vec_body, 0, unroll=8)
                return 0

            lax.fori_loop(0, _CH, row_body, 0)

        def start_out(c, b):
            row0 = base + c * _CH
            pltpu.async_copy(o_v.at[b], out_hbm.at[pl.ds(row0, _CH)], souts[b])

        def wait_out(c, b):
            row0 = base + c * _CH
            pltpu.make_async_copy(o_v.at[b], out_hbm.at[pl.ds(row0, _CH)], souts[b]).wait()

        start_in(0, 0)
        start_in(1, 1)

        def outer(t, _):
            for b in range(2):
                c = t * 2 + b
                wait_in(c, b)

                @pl.when(c >= 2)
                def _():
                    wait_out(c - 2, b)

                compute(c, b)
                start_out(c, b)

                @pl.when(c + 2 < nch)
                def _():
                    start_in(c + 2, b)
            return 0

        lax.fori_loop(0, nch // 2, outer, 0)
        wait_out(nch - 2, 0)
        wait_out(nch - 1, 1)

    return mix


def kernel(hidden_states, expert_outputs, expert_weights, expert_indices):
    K, B, S, H = expert_outputs.shape
    N = B * S
    e = expert_outputs.reshape(K, N, H)
    wb = jnp.broadcast_to(
        expert_weights.reshape(N, K)[:, :, None], (N, K, _L)
    ).reshape(N * K * _L)
    out = _sc_mix(N, H)(e, wb)
    return out.reshape(B, S, H)


# speedup vs baseline: 1.5185x; 1.5185x over previous
import functools

import jax
import jax.numpy as jnp
from jax import lax
from jax.experimental import pallas as pl
from jax.experimental.pallas import tpu as pltpu
from jax.experimental.pallas import tpu_sc as plsc

_NC = 2
_NS = 16
_NW = _NC * _NS
_L = 16

_CH = 8


def _sc_mix(n_rows, h, row_off=0):
    rpw = n_rows // _NW
    nch = rpw // _CH
    mesh = plsc.VectorSubcoreMesh(core_axis_name="c", subcore_axis_name="s")

    @functools.partial(
        pl.kernel,
        mesh=mesh,
        out_type=jax.ShapeDtypeStruct((n_rows, h), jnp.float32),
        scratch_types=[
            pltpu.VMEM((2, _CH, h), jnp.float32),
            pltpu.VMEM((2, _CH, h), jnp.float32),
            pltpu.VMEM((2, _CH, h), jnp.float32),
            pltpu.VMEM((rpw * 2 * _L,), jnp.float32),
            pltpu.SemaphoreType.DMA,
            pltpu.SemaphoreType.DMA,
            pltpu.SemaphoreType.DMA,
            pltpu.SemaphoreType.DMA,
        ],
    )
    def mix(e_hbm, w_hbm, out_hbm, e0_v, e1_v, o_v, w_v,
            sin0, sin1, sout0, sout1):
        wid = lax.axis_index("s") * _NC + lax.axis_index("c")
        base = wid * rpw
        sins = (sin0, sin1)
        souts = (sout0, sout1)
        pltpu.sync_copy(w_hbm.at[pl.ds(base * 2 * _L, rpw * 2 * _L)], w_v)

        def start_in(c, b):
            row0 = row_off + base + c * _CH
            pltpu.async_copy(e_hbm.at[0, pl.ds(row0, _CH)], e0_v.at[b], sins[b])
            pltpu.async_copy(e_hbm.at[1, pl.ds(row0, _CH)], e1_v.at[b], sins[b])

        def wait_in(c, b):
            row0 = row_off + base + c * _CH
            pltpu.make_async_copy(e_hbm.at[0, pl.ds(row0, _CH)], e0_v.at[b], sins[b]).wait()
            pltpu.make_async_copy(e_hbm.at[1, pl.ds(row0, _CH)], e1_v.at[b], sins[b]).wait()

        def compute(c, b):
            def row_body(r, _):
                woff = (c * _CH + r) * 2 * _L
                w0 = w_v[pl.ds(woff, _L)]
                w1 = w_v[pl.ds(woff + _L, _L)]

                def vec_body(j, _):
                    sl = pl.ds(j * _L, _L)
                    o_v[b, r, sl] = e0_v[b, r, sl] * w0 + e1_v[b, r, sl] * w1
                    return 0

                lax.fori_loop(0, h // _L, vec_body, 0)
                return 0

            lax.fori_loop(0, _CH, row_body, 0)

        def start_out(c, b):
            row0 = base + c * _CH
            pltpu.async_copy(o_v.at[b], out_hbm.at[pl.ds(row0, _CH)], souts[b])

        def wait_out(c, b):
            row0 = base + c * _CH
            pltpu.make_async_copy(o_v.at[b], out_hbm.at[pl.ds(row0, _CH)], souts[b]).wait()

        start_in(0, 0)
        start_in(1, 1)

        def outer(t, _):
            for b in range(2):
                c = t * 2 + b
                wait_in(c, b)

                @pl.when(c >= 2)
                def _():
                    wait_out(c - 2, b)

                compute(c, b)
                start_out(c, b)

                @pl.when(c + 2 < nch)
                def _():
                    start_in(c + 2, b)
            return 0

        lax.fori_loop(0, nch // 2, outer, 0)
        wait_out(nch - 2, 0)
        wait_out(nch - 1, 1)

    return mix


_TC_ROWS = 512
_SC_FRAC_NUM, _SC_FRAC_DEN = 1, 4


def _tc_mix_body(e_ref, w_ref, o_ref):
    acc = e_ref[0] * w_ref[:, 0:1]
    for k in range(1, e_ref.shape[0]):
        acc = acc + e_ref[k] * w_ref[:, k : k + 1]
    o_ref[...] = acc


def kernel(hidden_states, expert_outputs, expert_weights, expert_indices):
    K, B, S, H = expert_outputs.shape
    N = B * S
    n_sc = (N * _SC_FRAC_NUM // _SC_FRAC_DEN) // (_NW * _CH * 2) * (_NW * _CH * 2)
    n_tc = N - n_sc
    e = expert_outputs.reshape(K, N, H)
    w = expert_weights.reshape(N, K)

    tc_out = pl.pallas_call(
        _tc_mix_body,
        grid=(n_tc // _TC_ROWS,),
        in_specs=[
            pl.BlockSpec((K, _TC_ROWS, H), lambda i: (0, i, 0)),
            pl.BlockSpec((_TC_ROWS, K), lambda i: (i, 0)),
        ],
        out_specs=pl.BlockSpec((_TC_ROWS, H), lambda i: (i, 0)),
        out_shape=jax.ShapeDtypeStruct((n_tc, H), jnp.float32),
    )(e, w)

    wb = jnp.broadcast_to(
        w[n_tc:, :, None], (n_sc, K, _L)
    ).reshape(n_sc * K * _L)
    sc_out = _sc_mix(n_sc, H, row_off=n_tc)(e, wb)

    out = jnp.concatenate([tc_out, sc_out], axis=0)
    return out.reshape(B, S, H)


# speedup vs baseline: 2.9043x; 1.9126x over previous
import functools

import jax
import jax.numpy as jnp
from jax import lax
from jax.experimental import pallas as pl
from jax.experimental.pallas import tpu as pltpu
from jax.experimental.pallas import tpu_sc as plsc

_NC = 2
_NS = 16
_NW = _NC * _NS
_L = 16

_CH = 8


def _sc_mix(n_rows, h, row_off=0):
    rpw = n_rows // _NW
    nch = rpw // _CH
    mesh = plsc.VectorSubcoreMesh(core_axis_name="c", subcore_axis_name="s")

    @functools.partial(
        pl.kernel,
        mesh=mesh,
        out_type=jax.ShapeDtypeStruct((n_rows, h), jnp.float32),
        scratch_types=[
            pltpu.VMEM((2, _CH, h), jnp.float32),
            pltpu.VMEM((2, _CH, h), jnp.float32),
            pltpu.VMEM((2, _CH, h), jnp.float32),
            pltpu.VMEM((rpw * 2 * _L,), jnp.float32),
            pltpu.SemaphoreType.DMA,
            pltpu.SemaphoreType.DMA,
            pltpu.SemaphoreType.DMA,
            pltpu.SemaphoreType.DMA,
        ],
    )
    def mix(e_hbm, w_hbm, out_hbm, e0_v, e1_v, o_v, w_v,
            sin0, sin1, sout0, sout1):
        wid = lax.axis_index("s") * _NC + lax.axis_index("c")
        base = wid * rpw
        sins = (sin0, sin1)
        souts = (sout0, sout1)
        pltpu.sync_copy(w_hbm.at[pl.ds(base * 2 * _L, rpw * 2 * _L)], w_v)

        def start_in(c, b):
            row0 = row_off + base + c * _CH
            pltpu.async_copy(e_hbm.at[0, pl.ds(row0, _CH)], e0_v.at[b], sins[b])
            pltpu.async_copy(e_hbm.at[1, pl.ds(row0, _CH)], e1_v.at[b], sins[b])

        def wait_in(c, b):
            row0 = row_off + base + c * _CH
            pltpu.make_async_copy(e_hbm.at[0, pl.ds(row0, _CH)], e0_v.at[b], sins[b]).wait()
            pltpu.make_async_copy(e_hbm.at[1, pl.ds(row0, _CH)], e1_v.at[b], sins[b]).wait()

        def compute(c, b):
            def row_body(r, _):
                woff = (c * _CH + r) * 2 * _L
                w0 = w_v[pl.ds(woff, _L)]
                w1 = w_v[pl.ds(woff + _L, _L)]

                def vec_body(j, _):
                    sl = pl.ds(j * _L, _L)
                    o_v[b, r, sl] = e0_v[b, r, sl] * w0 + e1_v[b, r, sl] * w1
                    return 0

                lax.fori_loop(0, h // _L, vec_body, 0)
                return 0

            lax.fori_loop(0, _CH, row_body, 0)

        def start_out(c, b):
            row0 = base + c * _CH
            pltpu.async_copy(o_v.at[b], out_hbm.at[pl.ds(row0, _CH)], souts[b])

        def wait_out(c, b):
            row0 = base + c * _CH
            pltpu.make_async_copy(o_v.at[b], out_hbm.at[pl.ds(row0, _CH)], souts[b]).wait()

        start_in(0, 0)
        start_in(1, 1)

        def outer(t, _):
            for b in range(2):
                c = t * 2 + b
                wait_in(c, b)

                @pl.when(c >= 2)
                def _():
                    wait_out(c - 2, b)

                compute(c, b)
                start_out(c, b)

                @pl.when(c + 2 < nch)
                def _():
                    start_in(c + 2, b)
            return 0

        lax.fori_loop(0, nch // 2, outer, 0)
        wait_out(nch - 2, 0)
        wait_out(nch - 1, 1)

    return mix


_TC_CH = 512


def _tc_stream(n, h, k):
    nch = n // _TC_CH

    def body(e_hbm, w_hbm, o_hbm, e_v, w_v, o_v, sin0, sin1, sout0, sout1):
        sins = (sin0, sin1)
        souts = (sout0, sout1)

        def start_in(c, b):
            row0 = c * _TC_CH
            pltpu.make_async_copy(
                e_hbm.at[:, pl.ds(row0, _TC_CH), :], e_v.at[b], sins[b]).start()
            pltpu.make_async_copy(
                w_hbm.at[pl.ds(row0, _TC_CH)], w_v.at[b], sins[b]).start()

        def wait_in(c, b):
            row0 = c * _TC_CH
            pltpu.make_async_copy(
                e_hbm.at[:, pl.ds(row0, _TC_CH), :], e_v.at[b], sins[b]).wait()
            pltpu.make_async_copy(
                w_hbm.at[pl.ds(row0, _TC_CH)], w_v.at[b], sins[b]).wait()

        def start_out(c, b):
            pltpu.make_async_copy(
                o_v.at[b], o_hbm.at[pl.ds(c * _TC_CH, _TC_CH)], souts[b]).start()

        def wait_out(c, b):
            pltpu.make_async_copy(
                o_v.at[b], o_hbm.at[pl.ds(c * _TC_CH, _TC_CH)], souts[b]).wait()

        def compute(b):
            acc = e_v[b, 0] * w_v[b][:, 0:1]
            for kk in range(1, k):
                acc = acc + e_v[b, kk] * w_v[b][:, kk : kk + 1]
            o_v[b] = acc

        start_in(0, 0)
        start_in(1, 1)

        def outer(t, _):
            for b in range(2):
                c = t * 2 + b
                wait_in(c, b)

                @pl.when(c >= 2)
                def _():
                    wait_out(c - 2, b)

                compute(b)
                start_out(c, b)

                @pl.when(c + 2 < nch)
                def _():
                    start_in(c + 2, b)
            return 0

        lax.fori_loop(0, nch // 2, outer, 0)
        wait_out(nch - 2, 0)
        wait_out(nch - 1, 1)

    return pl.pallas_call(
        body,
        in_specs=[
            pl.BlockSpec(memory_space=pl.ANY),
            pl.BlockSpec(memory_space=pl.ANY),
        ],
        out_specs=pl.BlockSpec(memory_space=pl.ANY),
        out_shape=jax.ShapeDtypeStruct((n, h), jnp.float32),
        scratch_shapes=[
            pltpu.VMEM((2, k, _TC_CH, h), jnp.float32),
            pltpu.VMEM((2, _TC_CH, k), jnp.float32),
            pltpu.VMEM((2, _TC_CH, h), jnp.float32),
            pltpu.SemaphoreType.DMA,
            pltpu.SemaphoreType.DMA,
            pltpu.SemaphoreType.DMA,
            pltpu.SemaphoreType.DMA,
        ],
    )


def kernel(hidden_states, expert_outputs, expert_weights, expert_indices):
    K, B, S, H = expert_outputs.shape
    N = B * S
    e = expert_outputs.reshape(K, N, H)
    w = expert_weights.reshape(N, K)
    out = _tc_stream(N, H, K)(e, w)
    return out.reshape(B, S, H)
